# Initial kernel scaffold; baseline (speedup 1.0000x reference)
import jax
import jax.numpy as jnp
from jax.experimental import pallas as pl


def kernel(x, t, emb_table, W, b):
    raise NotImplementedError("write your pallas kernel here")


# speedup vs baseline: 1.4806x; 1.4806x over previous
import functools

import jax
import jax.numpy as jnp
from jax import lax
from jax.experimental import pallas as pl
from jax.experimental.pallas import tpu as pltpu
from jax.experimental.pallas import tpu_sc as plsc

_N = 320000
_D = 128
_MAX_LEN = 240

_NUM_WORKERS = 32
_ROWS_PER_WORKER = _N // _NUM_WORKERS
_C = 80
_NITER = _ROWS_PER_WORKER // _C
_VREGS_PER_ROW = _D // 16


def _fuse_table_kernel(emb_ref, w_ref, b_ref, out_ref):
    out_ref[:, :] = (
        lax.dot_general(
            emb_ref[:, :], w_ref[:, :],
            dimension_numbers=(((1,), (1,)), ((), ())),
            preferred_element_type=jnp.float32,
        )
        + b_ref[:, :]
    )


def _sc_body(x_hbm, t_hbm, tab_hbm, out_hbm, idx_v, x_v, g_v, sem):
    wid = lax.axis_index("s") * 2 + lax.axis_index("c")
    w_base = wid * _ROWS_PER_WORKER

    def chunk(k, carry):
        base = w_base + k * _C
        pltpu.sync_copy(t_hbm.at[pl.ds(base, _C)], idx_v)
        pltpu.sync_copy(x_hbm.at[pl.ds(base, _C), :], x_v)
        pltpu.async_copy(tab_hbm.at[idx_v], g_v, sem).wait()

        def row(i, c2):
            for j in range(_VREGS_PER_ROW):
                sl = pl.ds(j * 16, 16)
                x_v[i, sl] = x_v[i, sl] + g_v[i, sl]
            return c2

        lax.fori_loop(0, _C, row, 0, unroll=False)
        pltpu.sync_copy(x_v, out_hbm.at[pl.ds(base, _C), :])
        return carry

    lax.fori_loop(0, _NITER, chunk, 0, unroll=False)


def kernel(x, t, emb_table, W, b):
    fused_table = pl.pallas_call(
        _fuse_table_kernel,
        out_shape=jax.ShapeDtypeStruct((_MAX_LEN, _D), jnp.float32),
    )(emb_table, W, b.reshape(1, _D))

    mesh = plsc.VectorSubcoreMesh(core_axis_name="c", subcore_axis_name="s")
    sc_gather_add = pl.kernel(
        _sc_body,
        out_type=jax.ShapeDtypeStruct((_N, _D), jnp.float32),
        mesh=mesh,
        scratch_types=[
            pltpu.VMEM((_C,), jnp.int32),
            pltpu.VMEM((_C, _D), jnp.float32),
            pltpu.VMEM((_C, _D), jnp.float32),
            pltpu.SemaphoreType.DMA,
        ],
    )
    return sc_gather_add(x, t, fused_table)
